# baseline (speedup 1.0000x reference)
import dataclasses
import functools

import jax
import jax.numpy as jnp
from jax import lax
from jax.experimental import pallas as pl
from jax.experimental.pallas import tpu as pltpu
from jax.experimental.pallas import tpu_sc as plsc

N = 10000
D = 128
E = 320000
STEPS = 3

NC = 2
NS = 16
NW = NC * NS
EPW = E // NW
K = 48
NWP = 216
EPWP = NWP * K
NRB = 4
NIB = 6
SEC = 12
RPT = 624
RTAIL = N - NS * RPT


def _sc_edge_scatter(m, src3, dst3, ew3, zeros_nd):
    mesh = plsc.VectorSubcoreMesh(core_axis_name="c", subcore_axis_name="s")
    cp = pltpu.CompilerParams()
    if "needs_layout_passes" in pltpu.CompilerParams.__dataclass_fields__:
        cp = dataclasses.replace(cp, needs_layout_passes=False)

    @functools.partial(
        pl.kernel,
        compiler_params=cp,
        out_type=jax.ShapeDtypeStruct((NC, N, D), jnp.float32),
        mesh=mesh,
        scratch_types=[
            pltpu.VMEM((NIB, K), jnp.int32),
            pltpu.VMEM((NIB, K), jnp.int32),
            pltpu.VMEM((NIB, K), jnp.float32),
            pltpu.VMEM((NRB, K, D), jnp.float32),
            pltpu.VMEM_SHARED((N, D), jnp.float32),
        ] + [pltpu.SemaphoreType.DMA] * (NIB + 2 * NRB),
    )
    def kern(m_hbm, src_hbm, dst_hbm, ew_hbm, z_hbm, out_hbm,
             src_v, dst_v, ew_v, rows_v, acc_sh, *sems):
        c = lax.axis_index("c")
        s = lax.axis_index("s")
        wid = s * NC + c
        isem = sems[:NIB]
        gsem = sems[NIB:NIB + NRB]
        ssem = sems[NIB + NRB:]

        def idx_copies(w, q):
            return (
                pltpu.make_async_copy(src_hbm.at[wid, w], src_v.at[q], isem[q]),
                pltpu.make_async_copy(dst_hbm.at[wid, w], dst_v.at[q], isem[q]),
                pltpu.make_async_copy(ew_hbm.at[wid, w], ew_v.at[q], isem[q]),
            )

        def gather_copy(r, q):
            return pltpu.make_async_copy(
                m_hbm.at[src_v.at[q]], rows_v.at[r], gsem[r])

        def scatter_copy(r, q):
            return pltpu.async_copy(
                rows_v.at[r], acc_sh.at[dst_v.at[q]], ssem[r], add=True)

        def scatter_wait(r, q):
            pltpu.make_async_copy(
                rows_v.at[r], acc_sh.at[dst_v.at[q]], ssem[r]).wait()

        pltpu.sync_copy(z_hbm.at[pl.ds(s * RPT, RPT)],
                        acc_sh.at[pl.ds(s * RPT, RPT)])

        @pl.when(s == NS - 1)
        def _ztail():
            pltpu.sync_copy(z_hbm.at[pl.ds(NS * RPT, RTAIL)],
                            acc_sh.at[pl.ds(NS * RPT, RTAIL)])

        for v in range(4):
            for cp_ in idx_copies(v, v):
                cp_.start()
        plsc.subcore_barrier()
        for cp_ in idx_copies(0, 0):
            cp_.wait()
        gather_copy(0, 0).start()
        for cp_ in idx_copies(1, 1):
            cp_.wait()
        gather_copy(1, 1).start()

        @pl.loop(0, NWP, step=SEC)
        def _sections(w0):
            for j in range(SEC):
                w = w0 + j
                r = j % NRB
                q = j % NIB

                gather_copy(r, q).wait()

                @pl.when(w >= 2)
                def _drain_prev():
                    scatter_wait((j - 2) % NRB, (j - 2) % NIB)

                @pl.when(w + 2 < NWP)
                def _issue_gather():
                    for cp_ in idx_copies(w + 2, (j + 2) % NIB):
                        cp_.wait()
                    gather_copy((j + 2) % NRB, (j + 2) % NIB).start()

                @pl.loop(0, K, step=8)
                def _rows(kk0):
                    for dk in range(8):
                        kk = kk0 + dk
                        wv = plsc.load_gather(
                            ew_v, [jnp.full((16,), q, jnp.int32),
                                   jnp.full((16,), kk, jnp.int32)])
                        for cc in range(D // 16):
                            sl = pl.ds(cc * 16, 16)
                            rows_v[r, kk, sl] = rows_v[r, kk, sl] * wv

                scatter_copy(r, q)

                @pl.when(w + 4 < NWP)
                def _prefetch_idx():
                    for cp_ in idx_copies(w + 4, (j + 4) % NIB):
                        cp_.start()

        scatter_wait((NWP - 2) % NRB, (NWP - 2) % NIB)
        scatter_wait((NWP - 1) % NRB, (NWP - 1) % NIB)

        plsc.subcore_barrier()
        pltpu.sync_copy(acc_sh.at[pl.ds(s * RPT, RPT)],
                        out_hbm.at[c, pl.ds(s * RPT, RPT)])

        @pl.when(s == NS - 1)
        def _otail():
            pltpu.sync_copy(acc_sh.at[pl.ds(NS * RPT, RTAIL)],
                            out_hbm.at[c, pl.ds(NS * RPT, RTAIL)])

    return kern(m, src3, dst3, ew3, zeros_nd)


BR = 2000


def _tc_first(x, w0):
    def body(x_ref, w_ref, m_ref):
        m_ref[...] = jnp.dot(x_ref[...], w_ref[...],
                             preferred_element_type=jnp.float32)

    return pl.pallas_call(
        body,
        grid=(N // BR,),
        in_specs=[pl.BlockSpec((BR, D), lambda i: (i, 0)),
                  pl.BlockSpec((D, D), lambda i: (0, 0))],
        out_specs=pl.BlockSpec((BR, D), lambda i: (i, 0)),
        out_shape=jax.ShapeDtypeStruct((N, D), jnp.float32),
    )(x, w0)


def _tc_gru(p, h, w_ih, w_hh, b_ih2, b_hh2, w_next):
    has_next = w_next is not None

    def body(p_ref, h_ref, wih_ref, whh_ref, bih_ref, bhh_ref, *rest):
        if has_next:
            wn_ref, h_out, m_out = rest
        else:
            (h_out,) = rest
        agg = p_ref[0] + p_ref[1]
        h = h_ref[...]
        gi = lax.dot_general(agg, wih_ref[...], (((1,), (1,)), ((), ())),
                             preferred_element_type=jnp.float32) + bih_ref[...]
        gh = lax.dot_general(h, whh_ref[...], (((1,), (1,)), ((), ())),
                             preferred_element_type=jnp.float32) + bhh_ref[...]
        r = jax.nn.sigmoid(gi[:, :D] + gh[:, :D])
        z = jax.nn.sigmoid(gi[:, D:2 * D] + gh[:, D:2 * D])
        n = jnp.tanh(gi[:, 2 * D:] + r * gh[:, 2 * D:])
        hn = (1.0 - z) * n + z * h
        h_out[...] = hn
        if has_next:
            m_out[...] = jnp.dot(hn, wn_ref[...],
                                 preferred_element_type=jnp.float32)

    out_shape = [jax.ShapeDtypeStruct((N, D), jnp.float32)]
    out_specs = [pl.BlockSpec((BR, D), lambda i: (i, 0))]
    in_specs = [
        pl.BlockSpec((2, BR, D), lambda i: (0, i, 0)),
        pl.BlockSpec((BR, D), lambda i: (i, 0)),
        pl.BlockSpec((3 * D, D), lambda i: (0, 0)),
        pl.BlockSpec((3 * D, D), lambda i: (0, 0)),
        pl.BlockSpec((1, 3 * D), lambda i: (0, 0)),
        pl.BlockSpec((1, 3 * D), lambda i: (0, 0)),
    ]
    args = [p, h, w_ih, w_hh, b_ih2, b_hh2]
    if has_next:
        out_shape.append(jax.ShapeDtypeStruct((N, D), jnp.float32))
        out_specs.append(pl.BlockSpec((BR, D), lambda i: (i, 0)))
        in_specs.append(pl.BlockSpec((D, D), lambda i: (0, 0)))
        args.append(w_next)
    res = pl.pallas_call(
        body,
        grid=(N // BR,),
        in_specs=in_specs,
        out_specs=out_specs,
        out_shape=out_shape,
    )(*args)
    return res if has_next else (res[0], None)


def kernel(x, edge_index, edge_weight, W, W_ih, W_hh, b_ih, b_hh):
    epad = EPWP - EPW
    padi = (jnp.arange(NW * epad, dtype=jnp.int32) % N).reshape(NW, epad)
    src3 = jnp.concatenate(
        [edge_index[0].reshape(NW, EPW), padi], axis=1).reshape(NW, NWP, K)
    dst3 = jnp.concatenate(
        [edge_index[1].reshape(NW, EPW), padi], axis=1).reshape(NW, NWP, K)
    ew3 = jnp.concatenate(
        [edge_weight.reshape(NW, EPW),
         jnp.zeros((NW, epad), jnp.float32)], axis=1).reshape(NW, NWP, K)
    zeros_nd = jnp.zeros((N, D), jnp.float32)
    b_ih2 = b_ih.reshape(1, 3 * D)
    b_hh2 = b_hh.reshape(1, 3 * D)

    h = x
    m = _tc_first(x, W[0])
    for i in range(STEPS):
        p = _sc_edge_scatter(m, src3, dst3, ew3, zeros_nd)
        w_next = W[i + 1] if i + 1 < STEPS else None
        h, m = _tc_gru(p, h, W_ih, W_hh, b_ih2, b_hh2, w_next)
    return h


# speedup vs baseline: 1.0908x; 1.0908x over previous
import dataclasses
import functools

import jax
import jax.numpy as jnp
from jax import lax
from jax.experimental import pallas as pl
from jax.experimental.pallas import tpu as pltpu
from jax.experimental.pallas import tpu_sc as plsc

N = 10000
D = 128
E = 320000
STEPS = 3

NC = 2
NS = 16
NW = NC * NS
EPW = E // NW
K = 56
NWP = 180
EPWP = NWP * K
NRB = 4
NIB = 6
SEC = 12
RPT = 624
RTAIL = N - NS * RPT


def _sc_edge_scatter(m, src3, dst3, ew3, zeros_nd):
    mesh = plsc.VectorSubcoreMesh(core_axis_name="c", subcore_axis_name="s")
    cp = pltpu.CompilerParams()
    if "needs_layout_passes" in pltpu.CompilerParams.__dataclass_fields__:
        cp = dataclasses.replace(cp, needs_layout_passes=False)

    @functools.partial(
        pl.kernel,
        compiler_params=cp,
        out_type=jax.ShapeDtypeStruct((NC, N, D), jnp.float32),
        mesh=mesh,
        scratch_types=[
            pltpu.VMEM((NIB, K), jnp.int32),
            pltpu.VMEM((NIB, K), jnp.int32),
            pltpu.VMEM((NIB, K), jnp.float32),
            pltpu.VMEM((NRB, K, D), jnp.float32),
            pltpu.VMEM_SHARED((N, D), jnp.float32),
        ] + [pltpu.SemaphoreType.DMA] * (NIB + 2 * NRB),
    )
    def kern(m_hbm, src_hbm, dst_hbm, ew_hbm, z_hbm, out_hbm,
             src_v, dst_v, ew_v, rows_v, acc_sh, *sems):
        c = lax.axis_index("c")
        s = lax.axis_index("s")
        wid = s * NC + c
        isem = sems[:NIB]
        gsem = sems[NIB:NIB + NRB]
        ssem = sems[NIB + NRB:]

        def idx_copies(w, q):
            return (
                pltpu.make_async_copy(src_hbm.at[wid, w], src_v.at[q], isem[q]),
                pltpu.make_async_copy(dst_hbm.at[wid, w], dst_v.at[q], isem[q]),
                pltpu.make_async_copy(ew_hbm.at[wid, w], ew_v.at[q], isem[q]),
            )

        def gather_copy(r, q):
            return pltpu.make_async_copy(
                m_hbm.at[src_v.at[q]], rows_v.at[r], gsem[r])

        def scatter_copy(r, q):
            return pltpu.async_copy(
                rows_v.at[r], acc_sh.at[dst_v.at[q]], ssem[r], add=True)

        def scatter_wait(r, q):
            pltpu.make_async_copy(
                rows_v.at[r], acc_sh.at[dst_v.at[q]], ssem[r]).wait()

        pltpu.sync_copy(z_hbm.at[pl.ds(s * RPT, RPT)],
                        acc_sh.at[pl.ds(s * RPT, RPT)])

        @pl.when(s == NS - 1)
        def _ztail():
            pltpu.sync_copy(z_hbm.at[pl.ds(NS * RPT, RTAIL)],
                            acc_sh.at[pl.ds(NS * RPT, RTAIL)])

        for v in range(4):
            for cp_ in idx_copies(v, v):
                cp_.start()
        plsc.subcore_barrier()
        for cp_ in idx_copies(0, 0):
            cp_.wait()
        gather_copy(0, 0).start()
        for cp_ in idx_copies(1, 1):
            cp_.wait()
        gather_copy(1, 1).start()

        @pl.loop(0, NWP, step=SEC)
        def _sections(w0):
            for j in range(SEC):
                w = w0 + j
                r = j % NRB
                q = j % NIB

                gather_copy(r, q).wait()

                @pl.when(w >= 2)
                def _drain_prev():
                    scatter_wait((j - 2) % NRB, (j - 2) % NIB)

                @pl.when(w + 2 < NWP)
                def _issue_gather():
                    for cp_ in idx_copies(w + 2, (j + 2) % NIB):
                        cp_.wait()
                    gather_copy((j + 2) % NRB, (j + 2) % NIB).start()

                @pl.loop(0, K, step=8)
                def _rows(kk0):
                    for dk in range(8):
                        kk = kk0 + dk
                        wv = plsc.load_gather(
                            ew_v, [jnp.full((16,), q, jnp.int32),
                                   jnp.full((16,), kk, jnp.int32)])
                        for cc in range(D // 16):
                            sl = pl.ds(cc * 16, 16)
                            rows_v[r, kk, sl] = rows_v[r, kk, sl] * wv

                scatter_copy(r, q)

                @pl.when(w + 4 < NWP)
                def _prefetch_idx():
                    for cp_ in idx_copies(w + 4, (j + 4) % NIB):
                        cp_.start()

        scatter_wait((NWP - 2) % NRB, (NWP - 2) % NIB)
        scatter_wait((NWP - 1) % NRB, (NWP - 1) % NIB)

        plsc.subcore_barrier()
        pltpu.sync_copy(acc_sh.at[pl.ds(s * RPT, RPT)],
                        out_hbm.at[c, pl.ds(s * RPT, RPT)])

        @pl.when(s == NS - 1)
        def _otail():
            pltpu.sync_copy(acc_sh.at[pl.ds(NS * RPT, RTAIL)],
                            out_hbm.at[c, pl.ds(NS * RPT, RTAIL)])

    return kern(m, src3, dst3, ew3, zeros_nd)


BR = 2000


def _tc_first(x, w0):
    def body(x_ref, w_ref, m_ref):
        m_ref[...] = jnp.dot(x_ref[...], w_ref[...],
                             preferred_element_type=jnp.float32)

    return pl.pallas_call(
        body,
        grid=(N // BR,),
        in_specs=[pl.BlockSpec((BR, D), lambda i: (i, 0)),
                  pl.BlockSpec((D, D), lambda i: (0, 0))],
        out_specs=pl.BlockSpec((BR, D), lambda i: (i, 0)),
        out_shape=jax.ShapeDtypeStruct((N, D), jnp.float32),
    )(x, w0)


def _tc_gru(p, h, w_ih, w_hh, b_ih2, b_hh2, w_next):
    has_next = w_next is not None

    def body(p_ref, h_ref, wih_ref, whh_ref, bih_ref, bhh_ref, *rest):
        if has_next:
            wn_ref, h_out, m_out = rest
        else:
            (h_out,) = rest
        agg = p_ref[0] + p_ref[1]
        h = h_ref[...]
        gi = lax.dot_general(agg, wih_ref[...], (((1,), (1,)), ((), ())),
                             preferred_element_type=jnp.float32) + bih_ref[...]
        gh = lax.dot_general(h, whh_ref[...], (((1,), (1,)), ((), ())),
                             preferred_element_type=jnp.float32) + bhh_ref[...]
        r = jax.nn.sigmoid(gi[:, :D] + gh[:, :D])
        z = jax.nn.sigmoid(gi[:, D:2 * D] + gh[:, D:2 * D])
        n = jnp.tanh(gi[:, 2 * D:] + r * gh[:, 2 * D:])
        hn = (1.0 - z) * n + z * h
        h_out[...] = hn
        if has_next:
            m_out[...] = jnp.dot(hn, wn_ref[...],
                                 preferred_element_type=jnp.float32)

    out_shape = [jax.ShapeDtypeStruct((N, D), jnp.float32)]
    out_specs = [pl.BlockSpec((BR, D), lambda i: (i, 0))]
    in_specs = [
        pl.BlockSpec((2, BR, D), lambda i: (0, i, 0)),
        pl.BlockSpec((BR, D), lambda i: (i, 0)),
        pl.BlockSpec((3 * D, D), lambda i: (0, 0)),
        pl.BlockSpec((3 * D, D), lambda i: (0, 0)),
        pl.BlockSpec((1, 3 * D), lambda i: (0, 0)),
        pl.BlockSpec((1, 3 * D), lambda i: (0, 0)),
    ]
    args = [p, h, w_ih, w_hh, b_ih2, b_hh2]
    if has_next:
        out_shape.append(jax.ShapeDtypeStruct((N, D), jnp.float32))
        out_specs.append(pl.BlockSpec((BR, D), lambda i: (i, 0)))
        in_specs.append(pl.BlockSpec((D, D), lambda i: (0, 0)))
        args.append(w_next)
    res = pl.pallas_call(
        body,
        grid=(N // BR,),
        in_specs=in_specs,
        out_specs=out_specs,
        out_shape=out_shape,
    )(*args)
    return res if has_next else (res[0], None)


def kernel(x, edge_index, edge_weight, W, W_ih, W_hh, b_ih, b_hh):
    epad = EPWP - EPW
    padi = (jnp.arange(NW * epad, dtype=jnp.int32) % N).reshape(NW, epad)
    src3 = jnp.concatenate(
        [edge_index[0].reshape(NW, EPW), padi], axis=1).reshape(NW, NWP, K)
    dst3 = jnp.concatenate(
        [edge_index[1].reshape(NW, EPW), padi], axis=1).reshape(NW, NWP, K)
    ew3 = jnp.concatenate(
        [edge_weight.reshape(NW, EPW),
         jnp.zeros((NW, epad), jnp.float32)], axis=1).reshape(NW, NWP, K)
    zeros_nd = jnp.zeros((N, D), jnp.float32)
    b_ih2 = b_ih.reshape(1, 3 * D)
    b_hh2 = b_hh.reshape(1, 3 * D)

    h = x
    m = _tc_first(x, W[0])
    for i in range(STEPS):
        p = _sc_edge_scatter(m, src3, dst3, ew3, zeros_nd)
        w_next = W[i + 1] if i + 1 < STEPS else None
        h, m = _tc_gru(p, h, W_ih, W_hh, b_ih2, b_hh2, w_next)
    return h
